# baseline (speedup 1.0000x reference)
import functools

import jax
import jax.numpy as jnp
from jax import lax
from jax.experimental import pallas as pl
from jax.experimental.pallas import tpu as pltpu
from jax.experimental.pallas import tpu_sc as plsc

N_VOCAB = 1000000
N_EMBED = 32
BATCH = 16384
ROW_W = 128
PACK = ROW_W // N_EMBED

_info = plsc.get_sparse_core_info()
_NC = _info.num_cores
_NS = _info.num_subcores
_NW = _NC * _NS
_B_PER_W = BATCH // _NW
_CHUNK = 128
_N_CHUNK = _B_PER_W // _CHUNK
_GROUPS = _CHUNK // 16


def _extract(idx_v, buf, compact, chunk_base):
    def body(g, _):
        lane = lax.iota(jnp.int32, 16)
        local = g * 16 + lane
        glob = chunk_base + local
        v = idx_v[pl.ds(chunk_base + g * 16, 16)]
        sub = (v & (PACK - 1)) * N_EMBED
        for j in range(N_EMBED):
            col = sub + j
            val = plsc.load_gather(buf, [local, col])
            jcol = lax.full((16,), j, jnp.int32)
            plsc.store_scatter(compact, [glob, jcol], val)
        return 0
    lax.fori_loop(0, _GROUPS, body, 0)


@functools.partial(
    pl.kernel,
    mesh=plsc.VectorSubcoreMesh(core_axis_name="c", subcore_axis_name="s"),
    out_type=[
        jax.ShapeDtypeStruct((BATCH, N_EMBED), jnp.float32),
        jax.ShapeDtypeStruct((BATCH, N_EMBED), jnp.float32),
        jax.ShapeDtypeStruct((BATCH, N_EMBED), jnp.float32),
    ],
    scratch_types=[
        pltpu.VMEM((_B_PER_W,), jnp.int32),
        pltpu.VMEM((_B_PER_W,), jnp.int32),
        pltpu.VMEM((_CHUNK, ROW_W), jnp.float32),
        pltpu.VMEM((_CHUNK, ROW_W), jnp.float32),
        pltpu.VMEM((_B_PER_W, N_EMBED), jnp.float32),
        pltpu.SemaphoreType.DMA,
        pltpu.SemaphoreType.DMA,
    ],
    compiler_params=pltpu.CompilerParams(needs_layout_passes=False),
)
def _gather3(in_hbm, out_hbm, self_hbm, iw_hbm, ow_hbm, w_hbm,
             o_in, o_out, o_self,
             idx_v, pidx_v, buf0, buf1, compact, sem0, sem1):
    wid = lax.axis_index("s") * _NC + lax.axis_index("c")
    base = wid * _B_PER_W
    bufs = (buf0, buf1)
    sems = (sem0, sem1)
    for tbl, idxh, outh in ((in_hbm, iw_hbm, o_in),
                            (out_hbm, ow_hbm, o_out),
                            (self_hbm, w_hbm, o_self)):
        pltpu.sync_copy(idxh.at[pl.ds(base, _B_PER_W)], idx_v)
        for g in range(_B_PER_W // 16):
            v = idx_v[pl.ds(g * 16, 16)]
            pidx_v[pl.ds(g * 16, 16)] = lax.shift_right_logical(v, 2)

        def fire(j):
            return pltpu.async_copy(
                tbl.at[pidx_v.at[pl.ds(j * _CHUNK, _CHUNK)]],
                bufs[j % 2], sems[j % 2])

        cp = fire(0)
        for j in range(_N_CHUNK):
            nxt = fire(j + 1) if j + 1 < _N_CHUNK else None
            cp.wait()
            _extract(idx_v, bufs[j % 2], compact, j * _CHUNK)
            cp = nxt
        pltpu.sync_copy(compact, outh.at[pl.ds(base, _B_PER_W)])


def kernel(self_embed, in_embed, out_embed, input_words, output_words, words):
    iw = input_words.astype(jnp.int32)
    ow = output_words.astype(jnp.int32)
    w = words.astype(jnp.int32)
    in2 = in_embed.reshape(N_VOCAB // PACK, ROW_W)
    out2 = out_embed.reshape(N_VOCAB // PACK, ROW_W)
    self2 = self_embed.reshape(N_VOCAB // PACK, ROW_W)
    o_in, o_out, o_self = _gather3(in2, out2, self2, iw, ow, w)
    return (o_in, o_out, o_self)


# speedup vs baseline: 1.2426x; 1.2426x over previous
import functools

import jax
import jax.numpy as jnp
from jax.experimental import pallas as pl
from jax.experimental.pallas import tpu as pltpu

N_VOCAB = 1000000
N_EMBED = 32
BATCH = 16384
CHUNK = 512
GRID = BATCH // CHUNK


def _tc_body(idx1, idx2, idx3, t1, t2, t3, o1, o2, o3, sem):
    i = pl.program_id(0)
    copies = []
    for idx, tbl, out in ((idx1, t1, o1), (idx2, t2, o2), (idx3, t3, o3)):
        for r in range(CHUNK):
            s = idx[i * CHUNK + r]
            copies.append(pltpu.make_async_copy(tbl.at[s], out.at[r], sem))
    for c in copies:
        c.start()
    for c in copies:
        c.wait()


def kernel(self_embed, in_embed, out_embed, input_words, output_words, words):
    i1 = input_words.astype(jnp.int32)
    i2 = output_words.astype(jnp.int32)
    i3 = words.astype(jnp.int32)
    grid_spec = pltpu.PrefetchScalarGridSpec(
        num_scalar_prefetch=3,
        grid=(GRID,),
        in_specs=[
            pl.BlockSpec(memory_space=pltpu.MemorySpace.HBM),
            pl.BlockSpec(memory_space=pltpu.MemorySpace.HBM),
            pl.BlockSpec(memory_space=pltpu.MemorySpace.HBM),
        ],
        out_specs=[
            pl.BlockSpec((CHUNK, N_EMBED), lambda i, *_: (i, 0)),
            pl.BlockSpec((CHUNK, N_EMBED), lambda i, *_: (i, 0)),
            pl.BlockSpec((CHUNK, N_EMBED), lambda i, *_: (i, 0)),
        ],
        scratch_shapes=[pltpu.SemaphoreType.DMA],
    )
    o1, o2, o3 = pl.pallas_call(
        _tc_body,
        grid_spec=grid_spec,
        out_shape=[
            jax.ShapeDtypeStruct((BATCH, N_EMBED), jnp.float32),
            jax.ShapeDtypeStruct((BATCH, N_EMBED), jnp.float32),
            jax.ShapeDtypeStruct((BATCH, N_EMBED), jnp.float32),
        ],
        compiler_params=pltpu.CompilerParams(
            dimension_semantics=("arbitrary",)),
    )(i1, i2, i3, in_embed, out_embed, self_embed)
    return (o1, o2, o3)


# speedup vs baseline: 1.3598x; 1.0943x over previous
import functools

import jax
import jax.numpy as jnp
from jax import lax
from jax.experimental import pallas as pl
from jax.experimental.pallas import tpu as pltpu
from jax.experimental.pallas import tpu_sc as plsc

N_VOCAB = 1000000
N_EMBED = 32
BATCH = 16384

_info = plsc.get_sparse_core_info()
_NC = _info.num_cores
_NS = _info.num_subcores
_NW = _NC * _NS

SC_BATCH = 9216
TC_BATCH = BATCH - SC_BATCH
_B_PER_W = SC_BATCH // _NW
_UNROLL = 16
_NSEM = 4

TC_CHUNK = 512
TC_GRID = TC_BATCH // TC_CHUNK


@functools.partial(
    pl.kernel,
    mesh=plsc.VectorSubcoreMesh(core_axis_name="c", subcore_axis_name="s"),
    out_type=[
        jax.ShapeDtypeStruct((SC_BATCH, N_EMBED), jnp.float32),
        jax.ShapeDtypeStruct((SC_BATCH, N_EMBED), jnp.float32),
        jax.ShapeDtypeStruct((SC_BATCH, N_EMBED), jnp.float32),
    ],
    scratch_types=[
        pltpu.VMEM((_B_PER_W,), jnp.int32),
        pltpu.VMEM((_B_PER_W, N_EMBED), jnp.float32),
        pltpu.SemaphoreType.DMA,
        pltpu.SemaphoreType.DMA,
        pltpu.SemaphoreType.DMA,
        pltpu.SemaphoreType.DMA,
    ],
    compiler_params=pltpu.CompilerParams(needs_layout_passes=False),
)
def _sc_gather3(in_hbm, out_hbm, self_hbm, iw_hbm, ow_hbm, w_hbm,
                o_in, o_out, o_self,
                idx_v, rows_v, sem0, sem1, sem2, sem3):
    wid = lax.axis_index("s") * _NC + lax.axis_index("c")
    base = wid * _B_PER_W
    sems = (sem0, sem1, sem2, sem3)
    for tbl, idxh, outh in ((in_hbm, iw_hbm, o_in),
                            (out_hbm, ow_hbm, o_out),
                            (self_hbm, w_hbm, o_self)):
        pltpu.sync_copy(idxh.at[pl.ds(base, _B_PER_W)], idx_v)

        def fire(i, _):
            v = idx_v[pl.ds(i * _UNROLL, _UNROLL)]
            for u in range(_UNROLL):
                pltpu.async_copy(tbl.at[v[u]], rows_v.at[i * _UNROLL + u],
                                 sems[u % _NSEM])
            return 0

        lax.fori_loop(0, _B_PER_W // _UNROLL, fire, 0)
        for k in range(_NSEM):
            pltpu.make_async_copy(
                tbl.at[pl.ds(0, _B_PER_W // _NSEM)],
                rows_v.at[pl.ds(0, _B_PER_W // _NSEM)], sems[k]).wait()
        pltpu.sync_copy(rows_v, outh.at[pl.ds(base, _B_PER_W)])


def _tc_body(idx1, idx2, idx3, t1, t2, t3, o1, o2, o3, sem):
    i = pl.program_id(0)
    copies = []
    for idx, tbl, out in ((idx1, t1, o1), (idx2, t2, o2), (idx3, t3, o3)):
        for r in range(TC_CHUNK):
            s = idx[i * TC_CHUNK + r]
            copies.append(pltpu.make_async_copy(tbl.at[s], out.at[r], sem))
    for c in copies:
        c.start()
    for c in copies:
        c.wait()


def _tc_gather3(t1, t2, t3, i1, i2, i3):
    grid_spec = pltpu.PrefetchScalarGridSpec(
        num_scalar_prefetch=3,
        grid=(TC_GRID,),
        in_specs=[
            pl.BlockSpec(memory_space=pltpu.MemorySpace.HBM),
            pl.BlockSpec(memory_space=pltpu.MemorySpace.HBM),
            pl.BlockSpec(memory_space=pltpu.MemorySpace.HBM),
        ],
        out_specs=[
            pl.BlockSpec((TC_CHUNK, N_EMBED), lambda i, *_: (i, 0)),
            pl.BlockSpec((TC_CHUNK, N_EMBED), lambda i, *_: (i, 0)),
            pl.BlockSpec((TC_CHUNK, N_EMBED), lambda i, *_: (i, 0)),
        ],
        scratch_shapes=[pltpu.SemaphoreType.DMA],
    )
    return pl.pallas_call(
        _tc_body,
        grid_spec=grid_spec,
        out_shape=[
            jax.ShapeDtypeStruct((TC_BATCH, N_EMBED), jnp.float32),
            jax.ShapeDtypeStruct((TC_BATCH, N_EMBED), jnp.float32),
            jax.ShapeDtypeStruct((TC_BATCH, N_EMBED), jnp.float32),
        ],
        compiler_params=pltpu.CompilerParams(
            dimension_semantics=("arbitrary",)),
    )(i1, i2, i3, t1, t2, t3)


def kernel(self_embed, in_embed, out_embed, input_words, output_words, words):
    i1 = input_words.astype(jnp.int32)
    i2 = output_words.astype(jnp.int32)
    i3 = words.astype(jnp.int32)
    sc1, sc2, sc3 = _sc_gather3(in_embed, out_embed, self_embed,
                                i1[:SC_BATCH], i2[:SC_BATCH], i3[:SC_BATCH])
    tc1, tc2, tc3 = _tc_gather3(in_embed, out_embed, self_embed,
                                i1[SC_BATCH:], i2[SC_BATCH:], i3[SC_BATCH:])
    o1 = jnp.concatenate([sc1, tc1], axis=0)
    o2 = jnp.concatenate([sc2, tc2], axis=0)
    o3 = jnp.concatenate([sc3, tc3], axis=0)
    return (o1, o2, o3)


# speedup vs baseline: 1.3600x; 1.0002x over previous
import functools

import jax
import jax.numpy as jnp
from jax import lax
from jax.experimental import pallas as pl
from jax.experimental.pallas import tpu as pltpu
from jax.experimental.pallas import tpu_sc as plsc

N_VOCAB = 1000000
N_EMBED = 32
BATCH = 16384

_info = plsc.get_sparse_core_info()
_NC = _info.num_cores
_NS = _info.num_subcores
_NW = _NC * _NS

SC_BATCH = 9216
TC_BATCH = BATCH - SC_BATCH
_B_PER_W = SC_BATCH // _NW
_UNROLL = 16
_NSEM = 4

TC_CHUNK = 512
TC_GRID = TC_BATCH // TC_CHUNK


@functools.partial(
    pl.kernel,
    mesh=plsc.VectorSubcoreMesh(core_axis_name="c", subcore_axis_name="s"),
    out_type=[
        jax.ShapeDtypeStruct((SC_BATCH, N_EMBED), jnp.float32),
        jax.ShapeDtypeStruct((SC_BATCH, N_EMBED), jnp.float32),
        jax.ShapeDtypeStruct((SC_BATCH, N_EMBED), jnp.float32),
    ],
    scratch_types=[
        pltpu.VMEM((_B_PER_W,), jnp.int32),
        pltpu.VMEM((_B_PER_W, N_EMBED), jnp.float32),
        pltpu.SemaphoreType.DMA,
        pltpu.SemaphoreType.DMA,
        pltpu.SemaphoreType.DMA,
        pltpu.SemaphoreType.DMA,
    ],
    compiler_params=pltpu.CompilerParams(needs_layout_passes=False),
    cost_estimate=pl.CostEstimate(
        flops=0, transcendentals=0, bytes_accessed=800_000_000),
)
def _sc_gather3(in_hbm, out_hbm, self_hbm, iw_hbm, ow_hbm, w_hbm,
                o_in, o_out, o_self,
                idx_v, rows_v, sem0, sem1, sem2, sem3):
    wid = lax.axis_index("s") * _NC + lax.axis_index("c")
    base = wid * _B_PER_W
    sems = (sem0, sem1, sem2, sem3)
    for tbl, idxh, outh in ((in_hbm, iw_hbm, o_in),
                            (out_hbm, ow_hbm, o_out),
                            (self_hbm, w_hbm, o_self)):
        pltpu.sync_copy(idxh.at[pl.ds(base, _B_PER_W)], idx_v)

        def fire(i, _):
            v = idx_v[pl.ds(i * _UNROLL, _UNROLL)]
            for u in range(_UNROLL):
                pltpu.async_copy(tbl.at[v[u]], rows_v.at[i * _UNROLL + u],
                                 sems[u % _NSEM])
            return 0

        lax.fori_loop(0, _B_PER_W // _UNROLL, fire, 0)
        for k in range(_NSEM):
            pltpu.make_async_copy(
                tbl.at[pl.ds(0, _B_PER_W // _NSEM)],
                rows_v.at[pl.ds(0, _B_PER_W // _NSEM)], sems[k]).wait()
        pltpu.sync_copy(rows_v, outh.at[pl.ds(base, _B_PER_W)])


def _tc_body(idx1, idx2, idx3, t1, t2, t3, o1, o2, o3, sem):
    i = pl.program_id(0)
    copies = []
    for idx, tbl, out in ((idx1, t1, o1), (idx2, t2, o2), (idx3, t3, o3)):
        for r in range(TC_CHUNK):
            s = idx[i * TC_CHUNK + r]
            copies.append(pltpu.make_async_copy(tbl.at[s], out.at[r], sem))
    for c in copies:
        c.start()
    for c in copies:
        c.wait()


def _tc_gather3(t1, t2, t3, i1, i2, i3):
    grid_spec = pltpu.PrefetchScalarGridSpec(
        num_scalar_prefetch=3,
        grid=(TC_GRID,),
        in_specs=[
            pl.BlockSpec(memory_space=pltpu.MemorySpace.HBM),
            pl.BlockSpec(memory_space=pltpu.MemorySpace.HBM),
            pl.BlockSpec(memory_space=pltpu.MemorySpace.HBM),
        ],
        out_specs=[
            pl.BlockSpec((TC_CHUNK, N_EMBED), lambda i, *_: (i, 0)),
            pl.BlockSpec((TC_CHUNK, N_EMBED), lambda i, *_: (i, 0)),
            pl.BlockSpec((TC_CHUNK, N_EMBED), lambda i, *_: (i, 0)),
        ],
        scratch_shapes=[pltpu.SemaphoreType.DMA],
    )
    return pl.pallas_call(
        _tc_body,
        grid_spec=grid_spec,
        out_shape=[
            jax.ShapeDtypeStruct((TC_BATCH, N_EMBED), jnp.float32),
            jax.ShapeDtypeStruct((TC_BATCH, N_EMBED), jnp.float32),
            jax.ShapeDtypeStruct((TC_BATCH, N_EMBED), jnp.float32),
        ],
        compiler_params=pltpu.CompilerParams(
            dimension_semantics=("arbitrary",)),
    )(i1, i2, i3, t1, t2, t3)


def kernel(self_embed, in_embed, out_embed, input_words, output_words, words):
    i1 = input_words.astype(jnp.int32)
    i2 = output_words.astype(jnp.int32)
    i3 = words.astype(jnp.int32)
    sc1, sc2, sc3 = _sc_gather3(in_embed, out_embed, self_embed,
                                i1[:SC_BATCH], i2[:SC_BATCH], i3[:SC_BATCH])
    tc1, tc2, tc3 = _tc_gather3(in_embed, out_embed, self_embed,
                                i1[SC_BATCH:], i2[SC_BATCH:], i3[SC_BATCH:])
    o1 = jnp.concatenate([sc1, tc1], axis=0)
    o2 = jnp.concatenate([sc2, tc2], axis=0)
    o3 = jnp.concatenate([sc3, tc3], axis=0)
    return (o1, o2, o3)


# speedup vs baseline: 1.5002x; 1.1031x over previous
import functools

import jax
import jax.numpy as jnp
from jax import lax
from jax.experimental import pallas as pl
from jax.experimental.pallas import tpu as pltpu
from jax.experimental.pallas import tpu_sc as plsc

N_VOCAB = 1000000
N_EMBED = 32
BATCH = 16384

_info = plsc.get_sparse_core_info()
_NC = _info.num_cores
_NS = _info.num_subcores
_NW = _NC * _NS
_B_PER_W = BATCH // _NW
_UNROLL = 16
_NSEM = 4


@functools.partial(
    pl.kernel,
    mesh=plsc.VectorSubcoreMesh(core_axis_name="c", subcore_axis_name="s"),
    out_type=[
        jax.ShapeDtypeStruct((BATCH, N_EMBED), jnp.float32),
        jax.ShapeDtypeStruct((BATCH, N_EMBED), jnp.float32),
        jax.ShapeDtypeStruct((BATCH, N_EMBED), jnp.float32),
    ],
    scratch_types=[
        pltpu.VMEM((_B_PER_W,), jnp.int32),
        pltpu.VMEM((_B_PER_W, N_EMBED), jnp.float32),
        pltpu.SemaphoreType.DMA,
        pltpu.SemaphoreType.DMA,
        pltpu.SemaphoreType.DMA,
        pltpu.SemaphoreType.DMA,
    ],
    compiler_params=pltpu.CompilerParams(needs_layout_passes=False),
)
def _gather3(in_hbm, out_hbm, self_hbm, iw_hbm, ow_hbm, w_hbm,
             o_in, o_out, o_self,
             idx_v, rows_v, sem0, sem1, sem2, sem3):
    wid = lax.axis_index("s") * _NC + lax.axis_index("c")
    base = wid * _B_PER_W
    sems = (sem0, sem1, sem2, sem3)
    for tbl, idxh, outh in ((in_hbm, iw_hbm, o_in),
                            (out_hbm, ow_hbm, o_out),
                            (self_hbm, w_hbm, o_self)):
        pltpu.sync_copy(idxh.at[pl.ds(base, _B_PER_W)], idx_v)

        def fire(i, _):
            v = idx_v[pl.ds(i * _UNROLL, _UNROLL)]
            for u in range(_UNROLL):
                pltpu.async_copy(tbl.at[v[u]], rows_v.at[i * _UNROLL + u],
                                 sems[u % _NSEM])
            return 0

        lax.fori_loop(0, _B_PER_W // _UNROLL, fire, 0)
        for k in range(_NSEM):
            pltpu.make_async_copy(
                tbl.at[pl.ds(0, _B_PER_W // _NSEM)],
                rows_v.at[pl.ds(0, _B_PER_W // _NSEM)], sems[k]).wait()
        pltpu.sync_copy(rows_v, outh.at[pl.ds(base, _B_PER_W)])


def kernel(self_embed, in_embed, out_embed, input_words, output_words, words):
    iw = input_words.astype(jnp.int32)
    ow = output_words.astype(jnp.int32)
    w = words.astype(jnp.int32)
    o_in, o_out, o_self = _gather3(in_embed, out_embed, self_embed, iw, ow, w)
    return (o_in, o_out, o_self)
